# baseline (speedup 1.0000x reference)
import jax
import jax.numpy as jnp
from jax.experimental import pallas as pl
from jax.experimental.pallas import tpu as pltpu

_OBJECT_SCALE = 5.0
_NOOBJECT_SCALE = 1.0
_IGNORE_THRES = 0.6


def _region_body(nB, nA, nH, nW, HB):
    nsteps = nH // HB

    def body(out_ref, tgt_ref, anc_ref, loss_ref, r50_ref, r75_ref, aiou_ref,
             n2_ref, cnt_ref, raw_ref):
        s = pl.program_id(0)

        tt = jnp.transpose(tgt_ref[...])
        t0 = tt[:, 0:1]
        t1 = tt[:, 1:2]
        t2 = tt[:, 2:3]
        t3 = tt[:, 3:4]
        gt_x = t0 * nW
        gt_y = t1 * nH
        gt_w = t2 * nW
        gt_h = t3 * nH
        gxf = jnp.floor(gt_x)
        gyf = jnp.floor(gt_y)
        gx = gxf.astype(jnp.int32)
        gy = gyf.astype(jnp.int32)

        aw = [anc_ref[a, 0] for a in range(nA)]
        ah = [anc_ref[a, 1] for a in range(nA)]

        ratios = []
        for a in range(nA):
            inter = jnp.minimum(gt_w, aw[a]) * jnp.minimum(gt_h, ah[a])
            union = gt_w * gt_h + 1e-16 + aw[a] * ah[a] - inter
            ratios.append(inter / union)
        best = ratios[0]
        for a in range(1, nA):
            best = jnp.maximum(best, ratios[a])
        sels = []
        found = ratios[0] < ratios[0]
        for a in range(nA):
            is_best = jnp.logical_and(ratios[a] >= best, jnp.logical_not(found))
            sels.append(is_best)
            found = jnp.logical_or(found, is_best)

        b2x1 = gt_x - gt_w / 2
        b2x2 = gt_x + gt_w / 2
        b2y1 = gt_y - gt_h / 2
        b2y2 = gt_y + gt_h / 2

        lane = jax.lax.broadcasted_iota(jnp.int32, (1, nW), 1)
        celleq = lane == gx
        colf = lane.astype(jnp.float32)

        @pl.when(s == 0)
        def _init():
            n2_ref[...] = jnp.zeros_like(n2_ref)
            cnt_ref[...] = jnp.zeros_like(cnt_ref)
            raw_ref[...] = jnp.zeros_like(raw_ref)

        n2_map = jnp.zeros((nB, nW), jnp.float32)
        cnt_map = jnp.zeros((nB, nW), jnp.float32)
        raws = [jnp.zeros((nB, nW), jnp.float32) for _ in range(5)]

        for j in range(HB):
            rowv = s * HB + j
            rowmatch = gy == rowv
            rowf = rowv.astype(jnp.float32)
            rmask = jnp.logical_and(rowmatch, celleq)
            for a in range(nA):
                base = 5 * a
                xo = out_ref[base + 0, j]
                yo = out_ref[base + 1, j]
                wo = out_ref[base + 2, j]
                ho = out_ref[base + 3, j]
                co = out_ref[base + 4, j]
                x = 1.0 / (1.0 + jnp.exp(-xo))
                y = 1.0 / (1.0 + jnp.exp(-yo))
                conf = 1.0 / (1.0 + jnp.exp(-co))
                px = x + colf
                py = y + rowf
                pw = jnp.exp(wo) * aw[a]
                ph = jnp.exp(ho) * ah[a]

                hw = pw * 0.5
                hh = ph * 0.5
                b1x1 = px - hw
                b1x2 = px + hw
                b1y1 = py - hh
                b1y2 = py + hh
                ix1 = jnp.maximum(b1x1, b2x1)
                iy1 = jnp.maximum(b1y1, b2y1)
                ix2 = jnp.minimum(b1x2, b2x2)
                iy2 = jnp.minimum(b1y2, b2y2)
                inter = jnp.maximum(ix2 - ix1 + 1.0, 0.0) * jnp.maximum(iy2 - iy1 + 1.0, 0.0)
                a1 = (b1x2 - b1x1 + 1.0) * (b1y2 - b1y1 + 1.0)
                a2 = (b2x2 - b2x1 + 1.0) * (b2y2 - b2y1 + 1.0)
                noobj = inter <= _IGNORE_THRES * (a1 + a2 - inter + 1e-16)
                cm = jnp.where(noobj, conf, 0.0)
                n2_map = n2_map + cm * cm
                cnt_map = cnt_map + jnp.where(noobj, 1.0, 0.0)

                m = jnp.logical_and(rmask, sels[a])
                for c, slab in enumerate((xo, yo, wo, ho, co)):
                    raws[c] = raws[c] + jnp.where(m, slab, 0.0)

        n2_ref[...] = n2_ref[...] + n2_map
        cnt_ref[...] = cnt_ref[...] + cnt_map
        for c in range(5):
            raw_ref[c] = raw_ref[c] + raws[c]

        @pl.when(s == nsteps - 1)
        def _fin():
            rx = jnp.sum(raw_ref[0], axis=1, keepdims=True)
            ry = jnp.sum(raw_ref[1], axis=1, keepdims=True)
            rw = jnp.sum(raw_ref[2], axis=1, keepdims=True)
            rh = jnp.sum(raw_ref[3], axis=1, keepdims=True)
            rc = jnp.sum(raw_ref[4], axis=1, keepdims=True)

            a_w_best = jnp.zeros((nB, 1), jnp.float32)
            a_h_best = jnp.zeros((nB, 1), jnp.float32)
            for a in range(nA):
                a_w_best = a_w_best + jnp.where(sels[a], aw[a], 0.0)
                a_h_best = a_h_best + jnp.where(sels[a], ah[a], 0.0)

            x_obj = 1.0 / (1.0 + jnp.exp(-rx))
            y_obj = 1.0 / (1.0 + jnp.exp(-ry))
            conf_obj = 1.0 / (1.0 + jnp.exp(-rc))
            pw_obj = jnp.exp(rw) * a_w_best
            ph_obj = jnp.exp(rh) * a_h_best

            tx = gt_x - gxf
            ty = gt_y - gyf
            tw = jnp.log(gt_w / a_w_best + 1e-16)
            th = jnp.log(gt_h / a_h_best + 1e-16)
            scale = 2.0 - t2 * t3

            sq_x = (x_obj * scale - tx * scale) ** 2
            sq_y = (y_obj * scale - ty * scale) ** 2
            sq_w = (rw * scale - tw * scale) ** 2
            sq_h = (rh * scale - th * scale) ** 2
            sq_conf = (conf_obj - 1.0) ** 2

            px_o = x_obj + gxf
            py_o = y_obj + gyf
            hwo = pw_obj * 0.5
            hho = ph_obj * 0.5
            p1x1 = px_o - hwo
            p1x2 = px_o + hwo
            p1y1 = py_o - hho
            p1y2 = py_o + hho
            jx1 = jnp.maximum(p1x1, b2x1)
            jy1 = jnp.maximum(p1y1, b2y1)
            jx2 = jnp.minimum(p1x2, b2x2)
            jy2 = jnp.minimum(p1y2, b2y2)
            jinter = jnp.maximum(jx2 - jx1 + 1.0, 0.0) * jnp.maximum(jy2 - jy1 + 1.0, 0.0)
            ja1 = (p1x2 - p1x1 + 1.0) * (p1y2 - p1y1 + 1.0)
            ja2 = (b2x2 - b2x1 + 1.0) * (b2y2 - b2y1 + 1.0)
            jt = ja1 + ja2 - jinter + 1e-16
            iou_v = jinter / jt

            incl = jinter <= _IGNORE_THRES * jt
            corr_n2 = jnp.where(incl, conf_obj * conf_obj, 0.0)
            corr_cnt = jnp.where(incl, 1.0, 0.0)

            s_n2 = jnp.sum(n2_ref[...]) - jnp.sum(corr_n2)
            s_cnt = jnp.sum(cnt_ref[...]) - jnp.sum(corr_cnt)

            fnB = float(nB)
            n_noobj = jnp.maximum(s_cnt, 1.0)
            loss = (jnp.sum(sq_x) + jnp.sum(sq_y) + jnp.sum(sq_w)
                    + jnp.sum(sq_h) + _OBJECT_SCALE * jnp.sum(sq_conf)) / fnB \
                + _NOOBJECT_SCALE * s_n2 / n_noobj
            loss_ref[0] = loss
            r50_ref[0] = jnp.sum(jnp.where(iou_v > 0.5, 1.0, 0.0)) / fnB
            r75_ref[0] = jnp.sum(jnp.where(iou_v > 0.75, 1.0, 0.0)) / fnB
            aiou_ref[0] = jnp.sum(iou_v) / fnB

    return body


def kernel(output, targets, anchors):
    nB, C, nH, nW = output.shape
    nA = anchors.shape[0]
    HB = 4
    outT = jnp.transpose(output, (1, 2, 0, 3))
    body = _region_body(nB, nA, nH, nW, HB)
    outs = pl.pallas_call(
        body,
        grid=(nH // HB,),
        in_specs=[
            pl.BlockSpec((C, HB, nB, nW), lambda s: (0, s, 0, 0)),
            pl.BlockSpec((4, nB), lambda s: (0, 0)),
            pl.BlockSpec(memory_space=pltpu.SMEM),
        ],
        out_specs=[pl.BlockSpec(memory_space=pltpu.SMEM)] * 4,
        out_shape=[jax.ShapeDtypeStruct((1,), jnp.float32)] * 4,
        scratch_shapes=[
            pltpu.VMEM((nB, nW), jnp.float32),
            pltpu.VMEM((nB, nW), jnp.float32),
            pltpu.VMEM((5, nB, nW), jnp.float32),
        ],
    )(outT, targets.T, anchors)
    return tuple(o[0] for o in outs)


# speedup vs baseline: 1.6020x; 1.6020x over previous
import jax
import jax.numpy as jnp
from jax.experimental import pallas as pl
from jax.experimental.pallas import tpu as pltpu

_OBJECT_SCALE = 5.0
_NOOBJECT_SCALE = 1.0
_IGNORE_THRES = 0.6


def _region_body(nB, nA, nH, nW, HB):
    nsteps = nH // HB

    def body(out_ref, tgt_ref, anc_ref, loss_ref, r50_ref, r75_ref, aiou_ref,
             n2_ref, cnt_ref, raw_ref):
        s = pl.program_id(0)

        tt = jnp.transpose(tgt_ref[...])
        t0 = tt[:, 0:1]
        t1 = tt[:, 1:2]
        t2 = tt[:, 2:3]
        t3 = tt[:, 3:4]
        gt_x = t0 * nW
        gt_y = t1 * nH
        gt_w = t2 * nW
        gt_h = t3 * nH
        gxf = jnp.floor(gt_x)
        gyf = jnp.floor(gt_y)
        gx = gxf.astype(jnp.int32)
        gy = gyf.astype(jnp.int32)

        aw = [anc_ref[a, 0] for a in range(nA)]
        ah = [anc_ref[a, 1] for a in range(nA)]

        ratios = []
        for a in range(nA):
            inter = jnp.minimum(gt_w, aw[a]) * jnp.minimum(gt_h, ah[a])
            union = gt_w * gt_h + 1e-16 + aw[a] * ah[a] - inter
            ratios.append(inter / union)
        best = ratios[0]
        for a in range(1, nA):
            best = jnp.maximum(best, ratios[a])
        sels = []
        found = ratios[0] < ratios[0]
        for a in range(nA):
            is_best = jnp.logical_and(ratios[a] >= best, jnp.logical_not(found))
            sels.append(is_best)
            found = jnp.logical_or(found, is_best)

        b2x1 = gt_x - gt_w / 2
        b2x2 = gt_x + gt_w / 2
        b2y1 = gt_y - gt_h / 2
        b2y2 = gt_y + gt_h / 2

        lane = jax.lax.broadcasted_iota(jnp.int32, (1, nW), 1)
        celleq = lane == gx
        colf = lane.astype(jnp.float32)

        @pl.when(s == 0)
        def _init():
            n2_ref[...] = jnp.zeros_like(n2_ref)
            cnt_ref[...] = jnp.zeros_like(cnt_ref)
            raw_ref[...] = jnp.zeros_like(raw_ref)

        n2_map = jnp.zeros((nB, nW), jnp.float32)
        cnt_map = jnp.zeros((nB, nW), jnp.float32)
        raws = [jnp.zeros((nB, nW), jnp.float32) for _ in range(5)]

        for j in range(HB):
            rowv = s * HB + j
            rowmatch = gy == rowv
            rowf = rowv.astype(jnp.float32)
            rmask = jnp.logical_and(rowmatch, celleq)
            for a in range(nA):
                base = 5 * a
                xo = out_ref[base + 0, j]
                yo = out_ref[base + 1, j]
                wo = out_ref[base + 2, j]
                ho = out_ref[base + 3, j]
                co = out_ref[base + 4, j]
                x = 1.0 / (1.0 + jnp.exp(-xo))
                y = 1.0 / (1.0 + jnp.exp(-yo))
                conf = 1.0 / (1.0 + jnp.exp(-co))
                px = x + colf
                py = y + rowf
                pw = jnp.exp(wo) * aw[a]
                ph = jnp.exp(ho) * ah[a]

                hw = pw * 0.5
                hh = ph * 0.5
                b1x1 = px - hw
                b1x2 = px + hw
                b1y1 = py - hh
                b1y2 = py + hh
                ix1 = jnp.maximum(b1x1, b2x1)
                iy1 = jnp.maximum(b1y1, b2y1)
                ix2 = jnp.minimum(b1x2, b2x2)
                iy2 = jnp.minimum(b1y2, b2y2)
                inter = jnp.maximum(ix2 - ix1 + 1.0, 0.0) * jnp.maximum(iy2 - iy1 + 1.0, 0.0)
                a1 = (b1x2 - b1x1 + 1.0) * (b1y2 - b1y1 + 1.0)
                a2 = (b2x2 - b2x1 + 1.0) * (b2y2 - b2y1 + 1.0)
                noobj = inter <= _IGNORE_THRES * (a1 + a2 - inter + 1e-16)
                cm = jnp.where(noobj, conf, 0.0)
                n2_map = n2_map + cm * cm
                cnt_map = cnt_map + jnp.where(noobj, 1.0, 0.0)

                m = jnp.logical_and(rmask, sels[a])
                for c, slab in enumerate((xo, yo, wo, ho, co)):
                    raws[c] = raws[c] + jnp.where(m, slab, 0.0)

        n2_ref[...] = n2_ref[...] + n2_map
        cnt_ref[...] = cnt_ref[...] + cnt_map
        for c in range(5):
            raw_ref[c] = raw_ref[c] + raws[c]

        @pl.when(s == nsteps - 1)
        def _fin():
            rx = jnp.sum(raw_ref[0], axis=1, keepdims=True)
            ry = jnp.sum(raw_ref[1], axis=1, keepdims=True)
            rw = jnp.sum(raw_ref[2], axis=1, keepdims=True)
            rh = jnp.sum(raw_ref[3], axis=1, keepdims=True)
            rc = jnp.sum(raw_ref[4], axis=1, keepdims=True)

            a_w_best = jnp.zeros((nB, 1), jnp.float32)
            a_h_best = jnp.zeros((nB, 1), jnp.float32)
            for a in range(nA):
                a_w_best = a_w_best + jnp.where(sels[a], aw[a], 0.0)
                a_h_best = a_h_best + jnp.where(sels[a], ah[a], 0.0)

            x_obj = 1.0 / (1.0 + jnp.exp(-rx))
            y_obj = 1.0 / (1.0 + jnp.exp(-ry))
            conf_obj = 1.0 / (1.0 + jnp.exp(-rc))
            pw_obj = jnp.exp(rw) * a_w_best
            ph_obj = jnp.exp(rh) * a_h_best

            tx = gt_x - gxf
            ty = gt_y - gyf
            tw = jnp.log(gt_w / a_w_best + 1e-16)
            th = jnp.log(gt_h / a_h_best + 1e-16)
            scale = 2.0 - t2 * t3

            sq_x = (x_obj * scale - tx * scale) ** 2
            sq_y = (y_obj * scale - ty * scale) ** 2
            sq_w = (rw * scale - tw * scale) ** 2
            sq_h = (rh * scale - th * scale) ** 2
            sq_conf = (conf_obj - 1.0) ** 2

            px_o = x_obj + gxf
            py_o = y_obj + gyf
            hwo = pw_obj * 0.5
            hho = ph_obj * 0.5
            p1x1 = px_o - hwo
            p1x2 = px_o + hwo
            p1y1 = py_o - hho
            p1y2 = py_o + hho
            jx1 = jnp.maximum(p1x1, b2x1)
            jy1 = jnp.maximum(p1y1, b2y1)
            jx2 = jnp.minimum(p1x2, b2x2)
            jy2 = jnp.minimum(p1y2, b2y2)
            jinter = jnp.maximum(jx2 - jx1 + 1.0, 0.0) * jnp.maximum(jy2 - jy1 + 1.0, 0.0)
            ja1 = (p1x2 - p1x1 + 1.0) * (p1y2 - p1y1 + 1.0)
            ja2 = (b2x2 - b2x1 + 1.0) * (b2y2 - b2y1 + 1.0)
            jt = ja1 + ja2 - jinter + 1e-16
            iou_v = jinter / jt

            incl = jinter <= _IGNORE_THRES * jt
            corr_n2 = jnp.where(incl, conf_obj * conf_obj, 0.0)
            corr_cnt = jnp.where(incl, 1.0, 0.0)

            s_n2 = jnp.sum(n2_ref[...]) - jnp.sum(corr_n2)
            s_cnt = jnp.sum(cnt_ref[...]) - jnp.sum(corr_cnt)

            fnB = float(nB)
            n_noobj = jnp.maximum(s_cnt, 1.0)
            loss = (jnp.sum(sq_x) + jnp.sum(sq_y) + jnp.sum(sq_w)
                    + jnp.sum(sq_h) + _OBJECT_SCALE * jnp.sum(sq_conf)) / fnB \
                + _NOOBJECT_SCALE * s_n2 / n_noobj
            loss_ref[0] = loss
            r50_ref[0] = jnp.sum(jnp.where(iou_v > 0.5, 1.0, 0.0)) / fnB
            r75_ref[0] = jnp.sum(jnp.where(iou_v > 0.75, 1.0, 0.0)) / fnB
            aiou_ref[0] = jnp.sum(iou_v) / fnB

    return body


def kernel(output, targets, anchors):
    nB, C, nH, nW = output.shape
    nA = anchors.shape[0]
    HB = 26
    outT = jnp.transpose(output, (1, 2, 0, 3))
    body = _region_body(nB, nA, nH, nW, HB)
    outs = pl.pallas_call(
        body,
        grid=(nH // HB,),
        in_specs=[
            pl.BlockSpec((C, HB, nB, nW), lambda s: (0, s, 0, 0)),
            pl.BlockSpec((4, nB), lambda s: (0, 0)),
            pl.BlockSpec(memory_space=pltpu.SMEM),
        ],
        out_specs=[pl.BlockSpec(memory_space=pltpu.SMEM)] * 4,
        out_shape=[jax.ShapeDtypeStruct((1,), jnp.float32)] * 4,
        scratch_shapes=[
            pltpu.VMEM((nB, nW), jnp.float32),
            pltpu.VMEM((nB, nW), jnp.float32),
            pltpu.VMEM((5, nB, nW), jnp.float32),
        ],
    )(outT, targets.T, anchors)
    return tuple(o[0] for o in outs)
